# baseline (speedup 1.0000x reference)
import functools
import math

import jax
import jax.numpy as jnp
import numpy as np
from jax import lax
from jax.experimental import pallas as pl
from jax.experimental.pallas import tpu as pltpu
from jax.experimental.pallas import tpu_sc as plsc

MAX_LEN = 5000
DIM = 1024
N = 16384
B = 8
ZERO_ROW = MAX_LEN
SCALE = math.sqrt(DIM)

NUM_CORES = 2
NUM_SUBCORES = 16
NUM_WORKERS = NUM_CORES * NUM_SUBCORES
ROWS_PER_WORKER = N // NUM_WORKERS
CHUNK = 16
NUM_CHUNKS = ROWS_PER_WORKER // CHUNK
LANES = 16


def _pe_table() -> np.ndarray:
    pe = np.zeros((MAX_LEN + 1, DIM), dtype=np.float32)
    position = np.arange(0, MAX_LEN, dtype=np.float32)[:, None]
    div_term = np.exp(
        np.arange(0, DIM, 2, dtype=np.float32) * -(math.log(10000.0) / DIM))
    pe[:MAX_LEN, 0::2] = np.sin(position * div_term)
    pe[:MAX_LEN, 1::2] = np.cos(position * div_term)
    return pe


_PE = _pe_table()


def _index_body(mask_ref, out_ref):
    m = mask_ref[...]
    x = m
    for sh in (1, 2, 4, 8, 16, 32, 64):
        x = x + jnp.concatenate(
            [jnp.zeros((128, sh), jnp.int32), x[:, :-sh]], axis=1)
    row_tot = x[:, 127:128]
    y = row_tot
    for sh in (1, 2, 4, 8, 16, 32, 64):
        y = y + jnp.concatenate(
            [jnp.zeros((sh, 1), jnp.int32), y[:-sh, :]], axis=0)
    cs = x + (y - row_tot)
    rank = cs - 1
    col = lax.broadcasted_iota(jnp.int32, (128, 128), 1)
    bmod = col & 7
    cums = []
    running = jnp.zeros((), jnp.int32)
    starts = []
    for b in range(B):
        sl_b = jnp.sum(jnp.where(bmod == b, m, 0))
        starts.append(running)
        running = running + sl_b
        cums.append(running)
    batch = jnp.zeros((128, 128), jnp.int32)
    for b in range(B):
        batch = batch + (rank >= cums[b]).astype(jnp.int32)
    batch = jnp.minimum(batch, B - 1)
    start_sel = jnp.zeros((128, 128), jnp.int32)
    for b in range(B):
        start_sel = start_sel + jnp.where(batch == b, starts[b], 0)
    pos = rank - start_sel
    out_ref[...] = jnp.where(m > 0, pos, ZERO_ROW)


def _row_indices(mask_i32):
    return pl.pallas_call(
        _index_body,
        out_shape=jax.ShapeDtypeStruct((128, 128), jnp.int32),
    )(mask_i32)


def _sc_body(emb_hbm, idx_hbm, pe_hbm, out_hbm,
             eb0, eb1, pb0, pb1, ob0, ob1, ibuf,
             es0, es1, gs0, gs1, ss0, ss1):
    eb, pb, ob = (eb0, eb1), (pb0, pb1), (ob0, ob1)
    es, gs, ss = (es0, es1), (gs0, gs1), (ss0, ss1)
    wid = lax.axis_index("s") * NUM_CORES + lax.axis_index("c")
    base = wid * ROWS_PER_WORKER

    pltpu.sync_copy(idx_hbm.at[pl.ds(base, ROWS_PER_WORKER)], ibuf)

    def start_loads(ch, b):
        row0 = base + ch * CHUNK
        pltpu.async_copy(emb_hbm.at[pl.ds(row0, CHUNK)], eb[b], es[b])
        pltpu.async_copy(pe_hbm.at[ibuf.at[pl.ds(ch * CHUNK, CHUNK)]],
                         pb[b], gs[b])

    def step(ch, b):
        row0 = base + ch * CHUNK
        pltpu.make_async_copy(emb_hbm.at[pl.ds(row0, CHUNK)],
                              eb[b], es[b]).wait()
        pltpu.make_async_copy(pe_hbm.at[ibuf.at[pl.ds(ch * CHUNK, CHUNK)]],
                              pb[b], gs[b]).wait()

        @pl.when(ch >= 2)
        def _():
            pltpu.make_async_copy(ob[b], out_hbm.at[pl.ds(row0, CHUNK)],
                                  ss[b]).wait()

        def row_fn(r, carry):
            for c0 in range(0, DIM, LANES):
                e = eb[b][r, pl.ds(c0, LANES)]
                p = pb[b][r, pl.ds(c0, LANES)]
                ob[b][r, pl.ds(c0, LANES)] = e * SCALE + p
            return carry

        lax.fori_loop(0, CHUNK, row_fn, 0)
        pltpu.async_copy(ob[b], out_hbm.at[pl.ds(row0, CHUNK)], ss[b])

        @pl.when(ch + 2 < NUM_CHUNKS)
        def _():
            start_loads(ch + 2, b)

    start_loads(0, 0)
    start_loads(1, 1)

    def pair_fn(pair, carry):
        step(2 * pair, 0)
        step(2 * pair + 1, 1)
        return carry

    lax.fori_loop(0, NUM_CHUNKS // 2, pair_fn, 0)
    last0 = base + (NUM_CHUNKS - 2) * CHUNK
    last1 = base + (NUM_CHUNKS - 1) * CHUNK
    pltpu.make_async_copy(ob[0], out_hbm.at[pl.ds(last0, CHUNK)], ss[0]).wait()
    pltpu.make_async_copy(ob[1], out_hbm.at[pl.ds(last1, CHUNK)], ss[1]).wait()


@functools.cache
def _sc_apply():
    buf = pltpu.VMEM((CHUNK, DIM), jnp.float32)
    return pl.kernel(
        _sc_body,
        mesh=plsc.VectorSubcoreMesh(core_axis_name="c", subcore_axis_name="s"),
        out_type=jax.ShapeDtypeStruct((N, DIM), jnp.float32),
        scratch_types=[
            buf, buf, buf, buf, buf, buf,
            pltpu.VMEM((ROWS_PER_WORKER,), jnp.int32),
            pltpu.SemaphoreType.DMA, pltpu.SemaphoreType.DMA,
            pltpu.SemaphoreType.DMA, pltpu.SemaphoreType.DMA,
            pltpu.SemaphoreType.DMA, pltpu.SemaphoreType.DMA,
        ],
    )


def kernel(emb, position_mask):
    mask_i32 = position_mask.reshape(128, 128).astype(jnp.int32)
    idx = _row_indices(mask_i32).reshape(-1)
    emb_flat = emb.reshape(N, DIM)
    out_flat = _sc_apply()(emb_flat, idx, jnp.asarray(_PE))
    return out_flat.reshape(emb.shape)
